# initial kernel scaffold (unmeasured)
import jax
import jax.numpy as jnp
from jax import lax
from jax.experimental import pallas as pl
from jax.experimental.pallas import tpu as pltpu

N_DEV = 4
V_PER = 16384
N_IDX = 2048
D = 1024
CHUNK = N_IDX // N_DEV
GATHER_SEMS = 16


def kernel(table, idx):
    my_pos = lax.axis_index("i")
    off = (my_pos * V_PER).astype(jnp.int32)
    local_idx = jnp.clip(idx - off, 0, V_PER - 1).astype(jnp.int32)
    mask2d = ((idx >= off) & (idx < off + V_PER)).astype(jnp.float32)
    mask2d = mask2d.reshape(N_IDX, 1)

    def body(table_ref, lidx_ref, mask_ref, out_ref,
             rs_buf, send_sems, recv_sems, gather_sems):
        p = lax.axis_index("i")
        left = lax.rem(p - 1 + N_DEV, N_DEV)
        right = lax.rem(p + 1, N_DEV)

        def issue(n, carry):
            slot = lax.rem(n, GATHER_SEMS)

            @pl.when(n >= GATHER_SEMS)
            def _():
                pltpu.make_async_copy(
                    table_ref.at[pl.ds(0, 1), :],
                    out_ref.at[pl.ds(0, 1), :],
                    gather_sems.at[slot],
                ).wait()

            row = lidx_ref[n]
            pltpu.make_async_copy(
                table_ref.at[pl.ds(row, 1), :],
                out_ref.at[pl.ds(n, 1), :],
                gather_sems.at[slot],
            ).start()
            return carry

        lax.fori_loop(0, N_IDX, issue, 0)
        for slot in range(GATHER_SEMS):
            pltpu.make_async_copy(
                table_ref.at[pl.ds(0, 1), :],
                out_ref.at[pl.ds(0, 1), :],
                gather_sems.at[slot],
            ).wait()

        out_ref[...] = out_ref[...] * mask_ref[...]

        barrier_sem = pltpu.get_barrier_semaphore()
        for nbr in (left, right):
            pl.semaphore_signal(
                barrier_sem, inc=1,
                device_id=(nbr,), device_id_type=pl.DeviceIdType.MESH,
            )
        pl.semaphore_wait(barrier_sem, 2)

        for s in range(N_DEV - 1):
            send_chunk = lax.rem(p - s + N_DEV, N_DEV)
            recv_chunk = lax.rem(p - s - 1 + N_DEV, N_DEV)
            rdma = pltpu.make_async_remote_copy(
                src_ref=out_ref.at[pl.ds(send_chunk * CHUNK, CHUNK), :],
                dst_ref=rs_buf.at[s],
                send_sem=send_sems.at[s],
                recv_sem=recv_sems.at[s],
                device_id=(right,),
                device_id_type=pl.DeviceIdType.MESH,
            )
            rdma.start()
            rdma.wait()
            out_ref[pl.ds(recv_chunk * CHUNK, CHUNK), :] = (
                out_ref[pl.ds(recv_chunk * CHUNK, CHUNK), :] + rs_buf[s]
            )

        for s in range(N_DEV - 1):
            j = (N_DEV - 1) + s
            send_chunk = lax.rem(p + 1 - s + N_DEV, N_DEV)
            recv_chunk = lax.rem(p - s + N_DEV, N_DEV)
            send = pltpu.make_async_remote_copy(
                src_ref=out_ref.at[pl.ds(send_chunk * CHUNK, CHUNK), :],
                dst_ref=out_ref.at[pl.ds(send_chunk * CHUNK, CHUNK), :],
                send_sem=send_sems.at[j],
                recv_sem=recv_sems.at[j],
                device_id=(right,),
                device_id_type=pl.DeviceIdType.MESH,
            )
            send.start()
            recv = pltpu.make_async_remote_copy(
                src_ref=out_ref.at[pl.ds(send_chunk * CHUNK, CHUNK), :],
                dst_ref=out_ref.at[pl.ds(recv_chunk * CHUNK, CHUNK), :],
                send_sem=send_sems.at[j],
                recv_sem=recv_sems.at[j],
                device_id=(left,),
                device_id_type=pl.DeviceIdType.MESH,
            )
            recv.wait_recv()
            send.wait_send()

    return pl.pallas_call(
        body,
        out_shape=jax.ShapeDtypeStruct((N_IDX, D), jnp.float32),
        in_specs=[
            pl.BlockSpec(memory_space=pltpu.ANY),
            pl.BlockSpec(memory_space=pltpu.SMEM),
            pl.BlockSpec(memory_space=pltpu.VMEM),
        ],
        out_specs=pl.BlockSpec(memory_space=pltpu.VMEM),
        scratch_shapes=[
            pltpu.VMEM((N_DEV - 1, CHUNK, D), jnp.float32),
            pltpu.SemaphoreType.DMA((2 * (N_DEV - 1),)),
            pltpu.SemaphoreType.DMA((2 * (N_DEV - 1),)),
            pltpu.SemaphoreType.DMA((GATHER_SEMS,)),
        ],
        compiler_params=pltpu.CompilerParams(collective_id=0),
    )(table, local_idx, mask2d)


# baseline (device time: 264433 ns/iter reference)
import jax
import jax.numpy as jnp
from jax import lax
from jax.experimental import pallas as pl
from jax.experimental.pallas import tpu as pltpu

N_DEV = 4
V_PER = 16384
N_IDX = 2048
D = 1024
CHUNK = N_IDX // N_DEV
GATHER_SEMS = 16


def kernel(table, idx):
    my_pos = lax.axis_index("i")
    off = (my_pos * V_PER).astype(jnp.int32)
    local_idx = jnp.clip(idx - off, 0, V_PER - 1).astype(jnp.int32)
    mask2d = ((idx >= off) & (idx < off + V_PER)).astype(jnp.float32)
    mask2d = mask2d.reshape(N_IDX, 1)

    def body(table_ref, lidx_ref, mask_ref, out_ref,
             rs_buf, send_sems, recv_sems, gather_sems):
        p = lax.axis_index("i")
        left = lax.rem(p - 1 + N_DEV, N_DEV)
        right = lax.rem(p + 1, N_DEV)

        def issue(n, carry):
            slot = lax.rem(n, GATHER_SEMS)

            @pl.when(n >= GATHER_SEMS)
            def _():
                pltpu.make_async_copy(
                    table_ref.at[pl.ds(0, 1), :],
                    out_ref.at[pl.ds(0, 1), :],
                    gather_sems.at[slot],
                ).wait()

            row = lidx_ref[n]
            pltpu.make_async_copy(
                table_ref.at[pl.ds(row, 1), :],
                out_ref.at[pl.ds(n, 1), :],
                gather_sems.at[slot],
            ).start()
            return carry

        lax.fori_loop(0, N_IDX, issue, 0)
        for slot in range(GATHER_SEMS):
            pltpu.make_async_copy(
                table_ref.at[pl.ds(0, 1), :],
                out_ref.at[pl.ds(0, 1), :],
                gather_sems.at[slot],
            ).wait()

        out_ref[...] = out_ref[...] * mask_ref[...]

        barrier_sem = pltpu.get_barrier_semaphore()
        for nbr in (left, right):
            pl.semaphore_signal(
                barrier_sem, inc=1,
                device_id=(nbr,), device_id_type=pl.DeviceIdType.MESH,
            )
        pl.semaphore_wait(barrier_sem, 2)

        for s in range(N_DEV - 1):
            send_chunk = lax.rem(p - s + N_DEV, N_DEV)
            recv_chunk = lax.rem(p - s - 1 + N_DEV, N_DEV)
            rdma = pltpu.make_async_remote_copy(
                src_ref=out_ref.at[pl.ds(send_chunk * CHUNK, CHUNK), :],
                dst_ref=rs_buf.at[s],
                send_sem=send_sems.at[s],
                recv_sem=recv_sems.at[s],
                device_id=(right,),
                device_id_type=pl.DeviceIdType.MESH,
            )
            rdma.start()
            rdma.wait()
            out_ref[pl.ds(recv_chunk * CHUNK, CHUNK), :] = (
                out_ref[pl.ds(recv_chunk * CHUNK, CHUNK), :] + rs_buf[s]
            )

        for s in range(N_DEV - 1):
            j = (N_DEV - 1) + s
            send_chunk = lax.rem(p + 1 - s + N_DEV, N_DEV)
            recv_chunk = lax.rem(p - s + N_DEV, N_DEV)
            send = pltpu.make_async_remote_copy(
                src_ref=out_ref.at[pl.ds(send_chunk * CHUNK, CHUNK), :],
                dst_ref=out_ref.at[pl.ds(send_chunk * CHUNK, CHUNK), :],
                send_sem=send_sems.at[j],
                recv_sem=recv_sems.at[j],
                device_id=(right,),
                device_id_type=pl.DeviceIdType.MESH,
            )
            send.start()
            recv = pltpu.make_async_remote_copy(
                src_ref=out_ref.at[pl.ds(send_chunk * CHUNK, CHUNK), :],
                dst_ref=out_ref.at[pl.ds(recv_chunk * CHUNK, CHUNK), :],
                send_sem=send_sems.at[j],
                recv_sem=recv_sems.at[j],
                device_id=(left,),
                device_id_type=pl.DeviceIdType.MESH,
            )
            recv.wait_recv()
            send.wait_send()

    return pl.pallas_call(
        body,
        out_shape=jax.ShapeDtypeStruct((N_IDX, D), jnp.float32),
        in_specs=[
            pl.BlockSpec(memory_space=pl.ANY),
            pl.BlockSpec(memory_space=pltpu.SMEM),
            pl.BlockSpec(memory_space=pltpu.VMEM),
        ],
        out_specs=pl.BlockSpec(memory_space=pltpu.VMEM),
        scratch_shapes=[
            pltpu.VMEM((N_DEV - 1, CHUNK, D), jnp.float32),
            pltpu.SemaphoreType.DMA((2 * (N_DEV - 1),)),
            pltpu.SemaphoreType.DMA((2 * (N_DEV - 1),)),
            pltpu.SemaphoreType.DMA((GATHER_SEMS,)),
        ],
        compiler_params=pltpu.CompilerParams(collective_id=0),
    )(table, local_idx, mask2d)


# device time: 123462 ns/iter; 2.1418x vs baseline; 2.1418x over previous
import jax
import jax.numpy as jnp
from jax import lax
from jax.experimental import pallas as pl
from jax.experimental.pallas import tpu as pltpu

N_DEV = 4
V_PER = 16384
N_IDX = 2048
D = 1024
HALF = N_IDX // 2
CHUNK = HALF // N_DEV
GATHER_SEMS = 16


def kernel(table, idx):
    my_pos = lax.axis_index("i")
    off = (my_pos * V_PER).astype(jnp.int32)
    owned = (idx >= off) & (idx < off + V_PER)
    cnt = jnp.sum(owned.astype(jnp.int32)).reshape(1)
    ord_pos = jnp.argsort(jnp.where(owned, 0, 1).astype(jnp.int32)).astype(
        jnp.int32
    )
    gather_rows = jnp.clip(idx[ord_pos] - off, 0, V_PER - 1).astype(jnp.int32)

    def body(table_ref, pos_ref, rows_ref, cnt_ref, out_ref,
             rs_buf_r, rs_buf_l,
             r_send_sems, r_recv_sems, l_send_sems, l_recv_sems,
             gather_sems):
        p = lax.axis_index("i")
        left = lax.rem(p - 1 + N_DEV, N_DEV)
        right = lax.rem(p + 1, N_DEV)
        n_own = cnt_ref[0]

        out_ref[...] = jnp.zeros((N_IDX, D), jnp.float32)

        def issue(n, carry):
            slot = lax.rem(n, GATHER_SEMS)

            @pl.when(n >= GATHER_SEMS)
            def _():
                pltpu.make_async_copy(
                    table_ref.at[pl.ds(0, 1), :],
                    out_ref.at[pl.ds(0, 1), :],
                    gather_sems.at[slot],
                ).wait()

            pltpu.make_async_copy(
                table_ref.at[pl.ds(rows_ref[n], 1), :],
                out_ref.at[pl.ds(pos_ref[n], 1), :],
                gather_sems.at[slot],
            ).start()
            return carry

        lax.fori_loop(0, n_own, issue, 0)
        for slot in range(GATHER_SEMS):
            @pl.when(slot < n_own)
            def _():
                pltpu.make_async_copy(
                    table_ref.at[pl.ds(0, 1), :],
                    out_ref.at[pl.ds(0, 1), :],
                    gather_sems.at[slot],
                ).wait()

        barrier_sem = pltpu.get_barrier_semaphore()
        for nbr in (left, right):
            pl.semaphore_signal(
                barrier_sem, inc=1,
                device_id=(nbr,), device_id_type=pl.DeviceIdType.MESH,
            )
        pl.semaphore_wait(barrier_sem, 2)

        def r_rows(c):
            return pl.ds(c * CHUNK, CHUNK)

        def l_rows(c):
            return pl.ds(HALF + c * CHUNK, CHUNK)

        for s in range(N_DEV - 1):
            sc_r = lax.rem(p - s + N_DEV, N_DEV)
            rc_r = lax.rem(p - s - 1 + N_DEV, N_DEV)
            sc_l = lax.rem(p + s, N_DEV)
            rc_l = lax.rem(p + s + 1, N_DEV)
            rdma_r = pltpu.make_async_remote_copy(
                src_ref=out_ref.at[r_rows(sc_r), :],
                dst_ref=rs_buf_r.at[s],
                send_sem=r_send_sems.at[s],
                recv_sem=r_recv_sems.at[s],
                device_id=(right,),
                device_id_type=pl.DeviceIdType.MESH,
            )
            rdma_l = pltpu.make_async_remote_copy(
                src_ref=out_ref.at[l_rows(sc_l), :],
                dst_ref=rs_buf_l.at[s],
                send_sem=l_send_sems.at[s],
                recv_sem=l_recv_sems.at[s],
                device_id=(left,),
                device_id_type=pl.DeviceIdType.MESH,
            )
            rdma_r.start()
            rdma_l.start()
            rdma_r.wait_recv()
            out_ref[r_rows(rc_r), :] = out_ref[r_rows(rc_r), :] + rs_buf_r[s]
            rdma_l.wait_recv()
            out_ref[l_rows(rc_l), :] = out_ref[l_rows(rc_l), :] + rs_buf_l[s]
            rdma_r.wait_send()
            rdma_l.wait_send()

        for s in range(N_DEV - 1):
            j = (N_DEV - 1) + s
            sc_r = lax.rem(p + 1 - s + N_DEV, N_DEV)
            rc_r = lax.rem(p - s + N_DEV, N_DEV)
            sc_l = lax.rem(p - 1 + s + N_DEV, N_DEV)
            rc_l = lax.rem(p + s, N_DEV)
            send_r = pltpu.make_async_remote_copy(
                src_ref=out_ref.at[r_rows(sc_r), :],
                dst_ref=out_ref.at[r_rows(sc_r), :],
                send_sem=r_send_sems.at[j],
                recv_sem=r_recv_sems.at[j],
                device_id=(right,),
                device_id_type=pl.DeviceIdType.MESH,
            )
            send_l = pltpu.make_async_remote_copy(
                src_ref=out_ref.at[l_rows(sc_l), :],
                dst_ref=out_ref.at[l_rows(sc_l), :],
                send_sem=l_send_sems.at[j],
                recv_sem=l_recv_sems.at[j],
                device_id=(left,),
                device_id_type=pl.DeviceIdType.MESH,
            )
            send_r.start()
            send_l.start()
            recv_r = pltpu.make_async_remote_copy(
                src_ref=out_ref.at[r_rows(sc_r), :],
                dst_ref=out_ref.at[r_rows(rc_r), :],
                send_sem=r_send_sems.at[j],
                recv_sem=r_recv_sems.at[j],
                device_id=(left,),
                device_id_type=pl.DeviceIdType.MESH,
            )
            recv_l = pltpu.make_async_remote_copy(
                src_ref=out_ref.at[l_rows(sc_l), :],
                dst_ref=out_ref.at[l_rows(rc_l), :],
                send_sem=l_send_sems.at[j],
                recv_sem=l_recv_sems.at[j],
                device_id=(right,),
                device_id_type=pl.DeviceIdType.MESH,
            )
            recv_r.wait_recv()
            recv_l.wait_recv()
            send_r.wait_send()
            send_l.wait_send()

    return pl.pallas_call(
        body,
        out_shape=jax.ShapeDtypeStruct((N_IDX, D), jnp.float32),
        in_specs=[
            pl.BlockSpec(memory_space=pl.ANY),
            pl.BlockSpec(memory_space=pltpu.SMEM),
            pl.BlockSpec(memory_space=pltpu.SMEM),
            pl.BlockSpec(memory_space=pltpu.SMEM),
        ],
        out_specs=pl.BlockSpec(memory_space=pltpu.VMEM),
        scratch_shapes=[
            pltpu.VMEM((N_DEV - 1, CHUNK, D), jnp.float32),
            pltpu.VMEM((N_DEV - 1, CHUNK, D), jnp.float32),
            pltpu.SemaphoreType.DMA((2 * (N_DEV - 1),)),
            pltpu.SemaphoreType.DMA((2 * (N_DEV - 1),)),
            pltpu.SemaphoreType.DMA((2 * (N_DEV - 1),)),
            pltpu.SemaphoreType.DMA((2 * (N_DEV - 1),)),
            pltpu.SemaphoreType.DMA((GATHER_SEMS,)),
        ],
        compiler_params=pltpu.CompilerParams(collective_id=0),
    )(table, ord_pos, gather_rows, cnt)
